# baseline (speedup 1.0000x reference)
import functools

import jax
import jax.numpy as jnp
from jax import lax
from jax.experimental import pallas as pl
from jax.experimental.pallas import tpu as pltpu
from jax.experimental.pallas import tpu_sc as plsc

_NUM_VALS = 1001
_TABLE_PAD = 1024
_BATCH = 16384


def _table_body(embedT_ref, w1t_ref, b1_ref, w2_ref, b2_ref, out_ref):
    h = jnp.einsum("kr,kj->rj", embedT_ref[...], w1t_ref[...],
                   preferred_element_type=jnp.float32)
    h = jnp.maximum(h + b1_ref[...], 0.0)
    logits = jnp.sum(h * w2_ref[...], axis=1) + b2_ref[0, 0]
    out_ref[...] = jax.nn.sigmoid(logits)


def _compute_table(embed, W1, b1, W2, b2):
    return pl.pallas_call(
        _table_body,
        grid=(1,),
        in_specs=[
            pl.BlockSpec((32, _TABLE_PAD), lambda i: (0, 0)),
            pl.BlockSpec((32, 64), lambda i: (0, 0)),
            pl.BlockSpec((64,), lambda i: (0,)),
            pl.BlockSpec((1, 64), lambda i: (0, 0)),
            pl.BlockSpec((1, 1), lambda i: (0, 0)),
        ],
        out_specs=pl.BlockSpec((_TABLE_PAD,), lambda i: (0,)),
        out_shape=jax.ShapeDtypeStruct((_TABLE_PAD,), jnp.float32),
        compiler_params=pltpu.CompilerParams(
            fuse_transposed_lhs_in_matmul=True),
    )(embed.T, W1.T, b1, W2, b2[None, :])


@functools.cache
def _build_gather():
    info = plsc.get_sparse_core_info()
    nw = info.num_cores * info.num_subcores
    lanes = info.num_lanes
    chunk = _BATCH // nw
    mesh = plsc.VectorSubcoreMesh(core_axis_name="c", subcore_axis_name="s")

    @functools.partial(
        pl.kernel,
        out_type=jax.ShapeDtypeStruct((_BATCH,), jnp.float32),
        mesh=mesh,
        scratch_types=[
            pltpu.VMEM((chunk,), jnp.int32),
            pltpu.VMEM((_TABLE_PAD,), jnp.float32),
            pltpu.VMEM((chunk,), jnp.float32),
            pltpu.SemaphoreType.DMA,
            pltpu.SemaphoreType.DMA,
        ],
        compiler_params=pltpu.CompilerParams(
            needs_layout_passes=False, skip_device_barrier=True),
    )
    def gather_kernel(t_hbm, table_hbm, out_hbm, idx_v, table_v, out_v,
                      sem_t, sem_i):
        wid = lax.axis_index("s") * info.num_cores + lax.axis_index("c")
        base = wid * chunk
        cp_table = pltpu.async_copy(table_hbm, table_v, sem_t)
        cp_idx = pltpu.async_copy(t_hbm.at[pl.ds(base, chunk)], idx_v, sem_i)
        cp_table.wait()
        cp_idx.wait()

        def body(j, carry):
            idx = idx_v[pl.ds(j * lanes, lanes)]
            out_v[pl.ds(j * lanes, lanes)] = plsc.load_gather(table_v, [idx])
            return carry

        lax.fori_loop(0, chunk // lanes, body, 0)
        pltpu.sync_copy(out_v, out_hbm.at[pl.ds(base, chunk)])

    return gather_kernel


def kernel(t, embed, W1, b1, W2, b2):
    table = _compute_table(embed, W1, b1, W2, b2)
    return _build_gather()(t.astype(jnp.int32), table)


# speedup vs baseline: 1.0359x; 1.0359x over previous
import functools

import jax
import jax.numpy as jnp
from jax import lax
from jax.experimental import pallas as pl
from jax.experimental.pallas import tpu as pltpu
from jax.experimental.pallas import tpu_sc as plsc

_NUM_VALS = 1001
_TABLE_PAD = 1024
_BATCH = 16384


def _table_body(embedT_ref, w1t_ref, b1_ref, w2_ref, b2_ref, out_ref):
    hT = jnp.einsum("kj,kr->jr", w1t_ref[...], embedT_ref[...],
                    preferred_element_type=jnp.float32)
    hT = jnp.maximum(hT + b1_ref[...][:, None], 0.0)
    logits = jnp.dot(w2_ref[...], hT,
                     preferred_element_type=jnp.float32) + b2_ref[0, 0]
    sig = jax.nn.sigmoid(logits)
    for s in range(8):
        out_ref[pl.ds(s, 1), :] = sig[:, s * 128:(s + 1) * 128]


def _compute_table(embed, W1, b1, W2, b2):
    return pl.pallas_call(
        _table_body,
        grid=(1,),
        in_specs=[
            pl.BlockSpec((32, _TABLE_PAD), lambda i: (0, 0)),
            pl.BlockSpec((32, 64), lambda i: (0, 0)),
            pl.BlockSpec((64,), lambda i: (0,)),
            pl.BlockSpec((1, 64), lambda i: (0, 0)),
            pl.BlockSpec((1, 1), lambda i: (0, 0)),
        ],
        out_specs=pl.BlockSpec((8, 128), lambda i: (0, 0)),
        out_shape=jax.ShapeDtypeStruct((8, 128), jnp.float32),
        compiler_params=pltpu.CompilerParams(
            fuse_transposed_lhs_in_matmul=True),
    )(embed.T, W1.T, b1, W2, b2[None, :]).reshape(_TABLE_PAD)


@functools.cache
def _build_gather():
    info = plsc.get_sparse_core_info()
    nw = info.num_cores * info.num_subcores
    lanes = info.num_lanes
    chunk = _BATCH // nw
    mesh = plsc.VectorSubcoreMesh(core_axis_name="c", subcore_axis_name="s")

    @functools.partial(
        pl.kernel,
        out_type=jax.ShapeDtypeStruct((_BATCH,), jnp.float32),
        mesh=mesh,
        scratch_types=[
            pltpu.VMEM((chunk,), jnp.int32),
            pltpu.VMEM((_TABLE_PAD,), jnp.float32),
            pltpu.VMEM((chunk,), jnp.float32),
            pltpu.SemaphoreType.DMA,
            pltpu.SemaphoreType.DMA,
        ],
        compiler_params=pltpu.CompilerParams(needs_layout_passes=False),
    )
    def gather_kernel(t_hbm, table_hbm, out_hbm, idx_v, table_v, out_v,
                      sem_t, sem_i):
        wid = lax.axis_index("s") * info.num_cores + lax.axis_index("c")
        base = wid * chunk
        cp_table = pltpu.async_copy(table_hbm, table_v, sem_t)
        cp_idx = pltpu.async_copy(t_hbm.at[pl.ds(base, chunk)], idx_v, sem_i)
        cp_table.wait()
        cp_idx.wait()

        def body(j, carry):
            idx = idx_v[pl.ds(j * lanes, lanes)]
            out_v[pl.ds(j * lanes, lanes)] = plsc.load_gather(table_v, [idx])
            return carry

        lax.fori_loop(0, chunk // lanes, body, 0)
        pltpu.sync_copy(out_v, out_hbm.at[pl.ds(base, chunk)])

    return gather_kernel


def kernel(t, embed, W1, b1, W2, b2):
    table = _compute_table(embed, W1, b1, W2, b2)
    return _build_gather()(t.astype(jnp.int32), table)
